# baseline (speedup 1.0000x reference)
import functools

import jax
import jax.numpy as jnp
from jax import lax
from jax.experimental import pallas as pl
from jax.experimental.pallas import tpu as pltpu
from jax.experimental.pallas import tpu_sc as plsc

N = 320000
D = 128
S = 10000
NW = 32
SPW = 320
SPAD = NW * SPW
CHUNK = 128
NV = D // 16
NEG_INF = float("-inf")


def _make_kernel():
    mesh = plsc.VectorSubcoreMesh(core_axis_name="c", subcore_axis_name="s")

    @functools.partial(
        pl.kernel,
        out_type=jax.ShapeDtypeStruct((S, D), jnp.float32),
        mesh=mesh,
        scratch_types=[
            pltpu.VMEM((48,), jnp.int32),
            pltpu.VMEM((CHUNK, D), jnp.float32),
            pltpu.VMEM((CHUNK, D), jnp.float32),
            pltpu.VMEM((CHUNK + 16,), jnp.int32),
            pltpu.VMEM((CHUNK + 16,), jnp.int32),
            pltpu.VMEM((SPW, D), jnp.float32),
            pltpu.VMEM((SPW, D), jnp.float32),
            pltpu.SemaphoreType.DMA,
            pltpu.SemaphoreType.DMA,
            pltpu.SemaphoreType.DMA,
            pltpu.SemaphoreType.DMA,
        ],
    )
    def segmax(x_hbm, ids_hbm, rb_hbm, out_hbm, rb_v, rows0_v, rows1_v,
               sid0_v, sid1_v, acca_v, accb_v, rs0, rs1, is0, is1):
        rows_b = (rows0_v, rows1_v)
        sid_b = (sid0_v, sid1_v)
        wid = lax.axis_index("s") * 2 + lax.axis_index("c")
        pltpu.sync_copy(rb_hbm, rb_v.at[pl.ds(0, 40)])
        rb_vec = rb_v[pl.ds(wid, 16)]
        r0 = rb_vec[0]
        r1 = rb_vec[1]
        s0 = pl.multiple_of(wid * SPW, 8)
        rsem = (rs0, rs1)
        isem = (is0, is1)
        neg = jnp.full((16,), NEG_INF, jnp.float32)

        def init_body(si, _):
            for j in range(NV):
                acca_v[si, pl.ds(j * 16, 16)] = neg
                accb_v[si, pl.ds(j * 16, 16)] = neg
            return 0

        lax.fori_loop(0, SPW, init_body, 0)

        a0 = pl.multiple_of((r0 // 8) * 8, 8)
        nchunks = (r1 - a0 + CHUNK - 1) // CHUNK

        def chunk_base(k):
            nominal = a0 + k * CHUNK
            return nominal, pl.multiple_of(jnp.minimum(nominal, N - CHUNK), 8)

        def start_dma(k, b):
            _, base = chunk_base(k)
            pltpu.async_copy(x_hbm.at[pl.ds(base, CHUNK)], rows_b[b], rsem[b])
            pltpu.async_copy(ids_hbm.at[pl.ds(base, CHUNK)],
                             sid_b[b].at[pl.ds(0, CHUNK)], isem[b])

        def wait_dma(b):
            pltpu.make_async_copy(x_hbm.at[pl.ds(0, CHUNK)], rows_b[b],
                                  rsem[b]).wait()
            pltpu.make_async_copy(ids_hbm.at[pl.ds(0, CHUNK)],
                                  sid_b[b].at[pl.ds(0, CHUNK)], isem[b]).wait()

        def row_update(acc_ref, b, i, sid, prev, acc):
            same = sid == prev
            sl = sid - s0
            acc = tuple(
                jnp.maximum(
                    jnp.where(same, acc[j], neg),
                    rows_b[b][i, pl.ds(j * 16, 16)],
                )
                for j in range(NV)
            )
            for j in range(NV):
                acc_ref[sl, pl.ds(j * 16, 16)] = acc[j]
            return sid, acc

        def process_chunk(k, b, carry):
            nominal, base = chunk_base(k)
            lo = jnp.maximum(r0, nominal) - base
            hi = jnp.minimum(r1, base + CHUNK) - base
            g_lo = (lo + 15) // 16
            g_hi = hi // 16
            m_lo = jnp.minimum(g_lo * 16, hi)
            m_hi = jnp.maximum(g_hi * 16, m_lo)

            def scalar_body(i, carry):
                prev_a, acc_a, prev_b, acc_b = carry
                sid = sid_b[b][pl.ds(i, 16)][0]
                prev_a, acc_a = row_update(acca_v, b, i, sid, prev_a, acc_a)
                return prev_a, acc_a, prev_b, acc_b

            def group_body(g, carry):
                prev_a, acc_a, prev_b, acc_b = carry
                i0 = g * 16
                idv = sid_b[b][pl.ds(i0, 16)]
                for t in range(8):
                    prev_a, acc_a = row_update(
                        acca_v, b, i0 + t, idv[t], prev_a, acc_a)
                    prev_b, acc_b = row_update(
                        accb_v, b, i0 + 8 + t, idv[8 + t], prev_b, acc_b)
                return prev_a, acc_a, prev_b, acc_b

            carry = lax.fori_loop(lo, m_lo, scalar_body, carry)
            carry = lax.fori_loop(g_lo, g_hi, group_body, carry)
            carry = lax.fori_loop(m_hi, hi, scalar_body, carry)
            return carry

        zero_acc = tuple(neg for _ in range(NV))

        npairs = jnp.maximum((nchunks + 1) // 2, 1)
        start_dma(0, 0)

        def pair_body(p, carry):
            k = 2 * p
            start_dma(k + 1, 1)
            wait_dma(0)
            carry = process_chunk(k, 0, carry)

            @pl.when(p + 1 < npairs)
            def _():
                start_dma(k + 2, 0)

            wait_dma(1)
            return process_chunk(k + 1, 1, carry)

        lax.fori_loop(0, npairs, pair_body,
                      (jnp.int32(-1), zero_acc, jnp.int32(-1), zero_acc))

        def merge_body(si, _):
            for j in range(NV):
                acca_v[si, pl.ds(j * 16, 16)] = jnp.maximum(
                    acca_v[si, pl.ds(j * 16, 16)],
                    accb_v[si, pl.ds(j * 16, 16)],
                )
            return 0

        lax.fori_loop(0, SPW, merge_body, 0)

        @pl.when(wid < NW - 1)
        def _():
            pltpu.sync_copy(acca_v, out_hbm.at[pl.ds(s0, SPW)])

        @pl.when(wid == NW - 1)
        def _():
            pltpu.sync_copy(acca_v.at[pl.ds(0, S - (NW - 1) * SPW)],
                            out_hbm.at[pl.ds(s0, S - (NW - 1) * SPW)])

    return segmax


_segmax = _make_kernel()


@jax.jit
def kernel(x, batch):
    batch = batch.astype(jnp.int32)
    seg_bounds = jnp.arange(40, dtype=jnp.int32) * SPW
    row_bounds = jnp.sum(batch[:, None] < seg_bounds[None, :], axis=0,
                         dtype=jnp.int32)
    return _segmax(x, batch, row_bounds)


# speedup vs baseline: 1.0093x; 1.0093x over previous
import functools

import jax
import jax.numpy as jnp
from jax import lax
from jax.experimental import pallas as pl
from jax.experimental.pallas import tpu as pltpu
from jax.experimental.pallas import tpu_sc as plsc

N = 320000
D = 128
S = 10000
NW = 32
SPW = 320
SPAD = NW * SPW
CHUNK = 256
NV = D // 16
NEG_INF = float("-inf")


def _make_kernel():
    mesh = plsc.VectorSubcoreMesh(core_axis_name="c", subcore_axis_name="s")

    @functools.partial(
        pl.kernel,
        out_type=jax.ShapeDtypeStruct((S, D), jnp.float32),
        mesh=mesh,
        scratch_types=[
            pltpu.VMEM((48,), jnp.int32),
            pltpu.VMEM((CHUNK, D), jnp.float32),
            pltpu.VMEM((CHUNK, D), jnp.float32),
            pltpu.VMEM((CHUNK + 16,), jnp.int32),
            pltpu.VMEM((CHUNK + 16,), jnp.int32),
            pltpu.VMEM((SPW + 8, D), jnp.float32),
            pltpu.VMEM((D,), jnp.float32),
            pltpu.SMEM((8,), jnp.int32),
            pltpu.SemaphoreType.DMA,
            pltpu.SemaphoreType.DMA,
            pltpu.SemaphoreType.DMA,
            pltpu.SemaphoreType.DMA,
        ],
    )
    def segmax(x_hbm, ids_hbm, rb_hbm, out_hbm, rb_v, rows0_v, rows1_v,
               sid0_v, sid1_v, acc_v, accr_v, prev_s, rs0, rs1, is0, is1):
        rows_b = (rows0_v, rows1_v)
        sid_b = (sid0_v, sid1_v)
        wid = lax.axis_index("s") * 2 + lax.axis_index("c")
        pltpu.sync_copy(rb_hbm, rb_v.at[pl.ds(0, 40)])
        rb_vec = rb_v[pl.ds(wid, 16)]
        r0 = rb_vec[0]
        r1 = rb_vec[1]
        s0 = pl.multiple_of(wid * SPW, 8)
        rsem = (rs0, rs1)
        isem = (is0, is1)
        neg = jnp.full((16,), NEG_INF, jnp.float32)

        def init_body(si, _):
            for j in range(NV):
                acc_v[si, pl.ds(j * 16, 16)] = neg
            return 0

        lax.fori_loop(0, SPW + 8, init_body, 0)
        for j in range(NV):
            accr_v[pl.ds(j * 16, 16)] = neg
        prev_s[0] = jnp.int32(-1)

        def flush_slot(prev):
            return jnp.where(prev < s0, SPW, prev - s0)

        a0 = pl.multiple_of((r0 // 8) * 8, 8)
        nchunks = (r1 - a0 + CHUNK - 1) // CHUNK

        def chunk_base(k):
            nominal = a0 + k * CHUNK
            return nominal, pl.multiple_of(jnp.minimum(nominal, N - CHUNK), 8)

        def start_dma(k, b):
            _, base = chunk_base(k)
            pltpu.async_copy(x_hbm.at[pl.ds(base, CHUNK)], rows_b[b], rsem[b])
            pltpu.async_copy(ids_hbm.at[pl.ds(base, CHUNK)],
                             sid_b[b].at[pl.ds(0, CHUNK)], isem[b])

        def wait_dma(b):
            pltpu.make_async_copy(x_hbm.at[pl.ds(0, CHUNK)], rows_b[b],
                                  rsem[b]).wait()
            pltpu.make_async_copy(ids_hbm.at[pl.ds(0, CHUNK)],
                                  sid_b[b].at[pl.ds(0, CHUNK)], isem[b]).wait()

        def load_accr():
            return tuple(accr_v[pl.ds(j * 16, 16)] for j in range(NV))

        def store_accr(acc):
            for j in range(NV):
                accr_v[pl.ds(j * 16, 16)] = acc[j]

        def row_update(b, i, sid, prev, acc):
            same = sid == prev
            sl = sid - s0
            acc = tuple(
                jnp.maximum(
                    jnp.where(same, acc[j], neg),
                    rows_b[b][i, pl.ds(j * 16, 16)],
                )
                for j in range(NV)
            )
            for j in range(NV):
                acc_v[sl, pl.ds(j * 16, 16)] = acc[j]
            return sid, acc

        def process_chunk(k, b):
            nominal, base = chunk_base(k)
            lo = jnp.maximum(r0, nominal) - base
            hi = jnp.minimum(r1, base + CHUNK) - base
            g_lo = (lo + 15) // 16
            g_hi = hi // 16
            m_lo = jnp.minimum(g_lo * 16, hi)
            m_hi = jnp.maximum(g_hi * 16, m_lo)

            def scalar_body(i, _):
                prev = prev_s[0]
                sid = sid_b[b][pl.ds(i, 16)][0]
                acc = load_accr()
                fsl = flush_slot(prev)
                for j in range(NV):
                    acc_v[fsl, pl.ds(j * 16, 16)] = acc[j]
                sid, acc = row_update(b, i, sid, prev, acc)
                store_accr(acc)
                prev_s[0] = sid
                return 0

            def group_body(g, _):
                i0 = g * 16
                idv = sid_b[b][pl.ds(i0, 16)]
                prev = prev_s[0]
                uniform = idv[15] == prev

                @pl.when(uniform)
                def _():
                    for j in range(NV):
                        lvl = [rows_b[b][i0 + t, pl.ds(j * 16, 16)]
                               for t in range(16)]
                        while len(lvl) > 1:
                            lvl = [jnp.maximum(lvl[p], lvl[p + 1])
                                   for p in range(0, len(lvl), 2)]
                        accr_v[pl.ds(j * 16, 16)] = jnp.maximum(
                            accr_v[pl.ds(j * 16, 16)], lvl[0])

                @pl.when(jnp.logical_not(uniform))
                def _():
                    acc = load_accr()
                    fsl = flush_slot(prev)
                    for j in range(NV):
                        acc_v[fsl, pl.ds(j * 16, 16)] = acc[j]
                    p = prev
                    for t in range(16):
                        p, acc = row_update(b, i0 + t, idv[t], p, acc)
                    store_accr(acc)
                    prev_s[0] = p

                return 0

            lax.fori_loop(lo, m_lo, scalar_body, 0)
            lax.fori_loop(g_lo, g_hi, group_body, 0)
            lax.fori_loop(m_hi, hi, scalar_body, 0)

        npairs = jnp.maximum((nchunks + 1) // 2, 1)
        start_dma(0, 0)

        def pair_body(p, _):
            k = 2 * p
            start_dma(k + 1, 1)
            wait_dma(0)
            process_chunk(k, 0)

            @pl.when(p + 1 < npairs)
            def _():
                start_dma(k + 2, 0)

            wait_dma(1)
            process_chunk(k + 1, 1)
            return 0

        lax.fori_loop(0, npairs, pair_body, 0)

        acc = load_accr()
        fsl = flush_slot(prev_s[0])
        for j in range(NV):
            acc_v[fsl, pl.ds(j * 16, 16)] = acc[j]

        @pl.when(wid < NW - 1)
        def _():
            pltpu.sync_copy(acc_v.at[pl.ds(0, SPW)], out_hbm.at[pl.ds(s0, SPW)])

        @pl.when(wid == NW - 1)
        def _():
            pltpu.sync_copy(acc_v.at[pl.ds(0, S - (NW - 1) * SPW)],
                            out_hbm.at[pl.ds(s0, S - (NW - 1) * SPW)])

    return segmax


_segmax = _make_kernel()


@jax.jit
def kernel(x, batch):
    batch = batch.astype(jnp.int32)
    seg_bounds = jnp.arange(40, dtype=jnp.int32) * SPW
    row_bounds = jnp.sum(batch[:, None] < seg_bounds[None, :], axis=0,
                         dtype=jnp.int32)
    return _segmax(x, batch, row_bounds)


# speedup vs baseline: 1.0620x; 1.0522x over previous
import functools

import jax
import jax.numpy as jnp
from jax import lax
from jax.experimental import pallas as pl
from jax.experimental.pallas import tpu as pltpu
from jax.experimental.pallas import tpu_sc as plsc

N = 320000
D = 128
S = 10000
NW = 32
SPW = 320
SPAD = NW * SPW
CHUNK = 256
NV = D // 16
NEG_INF = float("-inf")


def _make_kernel():
    mesh = plsc.VectorSubcoreMesh(core_axis_name="c", subcore_axis_name="s")

    @functools.partial(
        pl.kernel,
        out_type=jax.ShapeDtypeStruct((S, D), jnp.float32),
        mesh=mesh,
        scratch_types=[
            pltpu.VMEM((48,), jnp.int32),
            pltpu.VMEM((CHUNK, D), jnp.float32),
            pltpu.VMEM((CHUNK, D), jnp.float32),
            pltpu.VMEM((CHUNK + 16,), jnp.int32),
            pltpu.VMEM((CHUNK + 16,), jnp.int32),
            pltpu.VMEM((SPW, D), jnp.float32),
            pltpu.SemaphoreType.DMA,
            pltpu.SemaphoreType.DMA,
            pltpu.SemaphoreType.DMA,
            pltpu.SemaphoreType.DMA,
        ],
    )
    def segmax(x_hbm, ids_hbm, rb_hbm, out_hbm, rb_v, rows0_v, rows1_v,
               sid0_v, sid1_v, acc_v, rs0, rs1, is0, is1):
        rows_b = (rows0_v, rows1_v)
        sid_b = (sid0_v, sid1_v)
        wid = lax.axis_index("s") * 2 + lax.axis_index("c")
        pltpu.sync_copy(rb_hbm, rb_v.at[pl.ds(0, 40)])
        rb_vec = rb_v[pl.ds(wid, 16)]
        r0 = rb_vec[0]
        r1 = rb_vec[1]
        s0 = pl.multiple_of(wid * SPW, 8)
        rsem = (rs0, rs1)
        isem = (is0, is1)

        def init_body(si, _):
            for j in range(NV):
                acc_v[si, pl.ds(j * 16, 16)] = jnp.full((16,), NEG_INF, jnp.float32)
            return 0

        lax.fori_loop(0, SPW, init_body, 0)

        a0 = pl.multiple_of((r0 // 8) * 8, 8)
        nchunks = (r1 - a0 + CHUNK - 1) // CHUNK

        def chunk_base(k):
            nominal = a0 + k * CHUNK
            return nominal, pl.multiple_of(jnp.minimum(nominal, N - CHUNK), 8)

        def start_dma(k, b):
            _, base = chunk_base(k)
            pltpu.async_copy(x_hbm.at[pl.ds(base, CHUNK)], rows_b[b], rsem[b])
            pltpu.async_copy(ids_hbm.at[pl.ds(base, CHUNK)],
                             sid_b[b].at[pl.ds(0, CHUNK)], isem[b])

        def wait_dma(b):
            pltpu.make_async_copy(x_hbm.at[pl.ds(0, CHUNK)], rows_b[b],
                                  rsem[b]).wait()
            pltpu.make_async_copy(ids_hbm.at[pl.ds(0, CHUNK)],
                                  sid_b[b].at[pl.ds(0, CHUNK)], isem[b]).wait()

        def row_update(b, i, sid, carry):
            prev, acc = carry
            same = sid == prev
            sl = sid - s0
            acc = tuple(
                jnp.maximum(
                    jnp.where(same, acc[j], NEG_INF),
                    rows_b[b][i, pl.ds(j * 16, 16)],
                )
                for j in range(NV)
            )
            for j in range(NV):
                acc_v[sl, pl.ds(j * 16, 16)] = acc[j]
            return sid, acc

        def process_chunk(k, b, carry):
            nominal, base = chunk_base(k)
            lo = jnp.maximum(r0, nominal) - base
            hi = jnp.minimum(r1, base + CHUNK) - base
            g_lo = (lo + 15) // 16
            g_hi = hi // 16
            m_lo = jnp.minimum(g_lo * 16, hi)
            m_hi = jnp.maximum(g_hi * 16, m_lo)

            def scalar_body(i, carry):
                sid = sid_b[b][pl.ds(i, 16)][0]
                return row_update(b, i, sid, carry)

            def group_body(g, carry):
                i0 = g * 16
                idv = sid_b[b][pl.ds(i0, 16)]
                for t in range(16):
                    carry = row_update(b, i0 + t, idv[t], carry)
                return carry

            carry = lax.fori_loop(lo, m_lo, scalar_body, carry)
            carry = lax.fori_loop(g_lo, g_hi, group_body, carry)
            carry = lax.fori_loop(m_hi, hi, scalar_body, carry)
            return carry

        zero_acc = tuple(jnp.full((16,), NEG_INF, jnp.float32) for _ in range(NV))

        npairs = jnp.maximum((nchunks + 1) // 2, 1)
        start_dma(0, 0)

        def pair_body(p, carry):
            k = 2 * p
            start_dma(k + 1, 1)
            wait_dma(0)
            carry = process_chunk(k, 0, carry)

            @pl.when(p + 1 < npairs)
            def _():
                start_dma(k + 2, 0)

            wait_dma(1)
            return process_chunk(k + 1, 1, carry)

        lax.fori_loop(0, npairs, pair_body, (jnp.int32(-1), zero_acc))

        @pl.when(wid < NW - 1)
        def _():
            pltpu.sync_copy(acc_v, out_hbm.at[pl.ds(s0, SPW)])

        @pl.when(wid == NW - 1)
        def _():
            pltpu.sync_copy(acc_v.at[pl.ds(0, S - (NW - 1) * SPW)],
                            out_hbm.at[pl.ds(s0, S - (NW - 1) * SPW)])

    return segmax


_segmax = _make_kernel()


@jax.jit
def kernel(x, batch):
    batch = batch.astype(jnp.int32)
    seg_bounds = jnp.arange(40, dtype=jnp.int32) * SPW
    row_bounds = jnp.sum(batch[:, None] < seg_bounds[None, :], axis=0,
                         dtype=jnp.int32)
    return _segmax(x, batch, row_bounds)
